# baseline (speedup 1.0000x reference)
import numpy as np

import jax
import jax.numpy as jnp
from jax.experimental import pallas as pl

B, C, H, W = 64, 768, 24, 24
E, K, P = 16, 2, 4
HP, WP = H // P, W // P
NPATCH = HP * WP
HW = H * W
R = P * E * P


def _router_kernel(x_ref, w_ref, mask_ref, poolL_ref, poolR_ref,
                   ew_ref, ei_ref, logits_ref):
    X = x_ref[0].astype(jnp.bfloat16)
    Wm = w_ref[...].astype(jnp.bfloat16)
    Z = jnp.dot(Wm, X, preferred_element_type=jnp.float32)
    hi = jax.lax.Precision.HIGHEST
    Z = Z * mask_ref[...]
    T = jnp.dot(poolL_ref[...], Z, preferred_element_type=jnp.float32, precision=hi)
    L = jnp.dot(T, poolR_ref[...], preferred_element_type=jnp.float32, precision=hi)
    Lt = L.T
    logits_ref[0] = Lt

    m = jnp.max(Lt, axis=1, keepdims=True)
    ex = jnp.exp(Lt - m)
    probs = ex / jnp.sum(ex, axis=1, keepdims=True)

    idx = jax.lax.broadcasted_iota(jnp.int32, (NPATCH, E), 1)
    p1 = jnp.max(probs, axis=1, keepdims=True)
    i1 = jnp.min(jnp.where(probs == p1, idx, E), axis=1, keepdims=True)
    probs2 = jnp.where(idx == i1, -1.0, probs)
    p2 = jnp.max(probs2, axis=1, keepdims=True)
    i2 = jnp.min(jnp.where(probs2 == p2, idx, E), axis=1, keepdims=True)

    s = p1 + p2 + 1e-9
    ew_ref[0] = jnp.concatenate([p1 / s, p2 / s], axis=1)
    ei_ref[0] = jnp.concatenate([i1, i2], axis=1).astype(jnp.int32)


def _constants():
    r = np.arange(R)[:, None]
    l = np.arange(HW)[None, :]
    mask = ((r // (E * P) == (l // W) % P) & (r % P == l % P)).astype(np.float32)
    e = np.arange(E)[:, None]
    rr = np.arange(R)[None, :]
    poolL = ((rr % (E * P)) // P == e).astype(np.float32)
    ll = np.arange(HW)[:, None]
    jj = np.arange(NPATCH)[None, :]
    poolR = ((ll // (P * W) == jj // WP)
             & ((ll % W) // P == jj % WP)).astype(np.float32)
    return mask, poolL, poolR


def kernel(x, spatial_shape, gate_w):
    del spatial_shape
    xr = x.reshape(B, C, HW)
    Wm = jnp.transpose(gate_w, (2, 0, 3, 1)).reshape(R, C)
    mask, poolL, poolR = _constants()

    out = pl.pallas_call(
        _router_kernel,
        grid=(B,),
        in_specs=[
            pl.BlockSpec((1, C, HW), lambda b: (b, 0, 0)),
            pl.BlockSpec((R, C), lambda b: (0, 0)),
            pl.BlockSpec((R, HW), lambda b: (0, 0)),
            pl.BlockSpec((E, R), lambda b: (0, 0)),
            pl.BlockSpec((HW, NPATCH), lambda b: (0, 0)),
        ],
        out_specs=[
            pl.BlockSpec((1, NPATCH, K), lambda b: (b, 0, 0)),
            pl.BlockSpec((1, NPATCH, K), lambda b: (b, 0, 0)),
            pl.BlockSpec((1, NPATCH, E), lambda b: (b, 0, 0)),
        ],
        out_shape=[
            jax.ShapeDtypeStruct((B, NPATCH, K), jnp.float32),
            jax.ShapeDtypeStruct((B, NPATCH, K), jnp.int32),
            jax.ShapeDtypeStruct((B, NPATCH, E), jnp.float32),
        ],
    )(xr, Wm, jnp.asarray(mask), jnp.asarray(poolL), jnp.asarray(poolR))

    expert_weights, expert_indices, router_logits = out
    return expert_weights, expert_indices, router_logits


# speedup vs baseline: 1.2898x; 1.2898x over previous
import numpy as np

import jax
import jax.numpy as jnp
from jax.experimental import pallas as pl

B, C, H, W = 64, 768, 24, 24
E, K, P = 16, 2, 4
HP, WP = H // P, W // P
NPATCH = HP * WP
HW = H * W
R = P * E * P


G = 8
NB = G * NPATCH


def _router_kernel(x_ref, w_ref, mask_ref, poolL_ref, poolR_ref,
                   ew_ref, ei_ref, logits_ref):
    Wm = w_ref[...].astype(jnp.bfloat16)
    mask = mask_ref[...]
    hi = jax.lax.Precision.HIGHEST
    Ls = []
    for g in range(G):
        X = x_ref[g].astype(jnp.bfloat16)
        Z = jnp.dot(Wm, X, preferred_element_type=jnp.float32)
        Z = Z * mask
        T = jnp.dot(poolL_ref[...], Z, preferred_element_type=jnp.float32, precision=hi)
        L = jnp.dot(T, poolR_ref[...], preferred_element_type=jnp.float32, precision=hi)
        Ls.append(L)
    Lbig = jnp.concatenate(Ls, axis=1)
    Lt = Lbig.T
    logits_ref[...] = Lt.reshape(G, NPATCH, E)

    m = jnp.max(Lt, axis=1, keepdims=True)
    ex = jnp.exp(Lt - m)
    probs = ex / jnp.sum(ex, axis=1, keepdims=True)

    idx = jax.lax.broadcasted_iota(jnp.int32, (NB, E), 1)
    p1 = jnp.max(probs, axis=1, keepdims=True)
    i1 = jnp.min(jnp.where(probs == p1, idx, E), axis=1, keepdims=True)
    probs2 = jnp.where(idx == i1, -1.0, probs)
    p2 = jnp.max(probs2, axis=1, keepdims=True)
    i2 = jnp.min(jnp.where(probs2 == p2, idx, E), axis=1, keepdims=True)

    s = p1 + p2 + 1e-9
    ew_ref[...] = jnp.concatenate([p1 / s, p2 / s], axis=1).reshape(G, NPATCH, K)
    ei_ref[...] = jnp.concatenate([i1, i2], axis=1).astype(jnp.int32).reshape(G, NPATCH, K)


def _constants():
    r = np.arange(R)[:, None]
    l = np.arange(HW)[None, :]
    mask = ((r // (E * P) == (l // W) % P) & (r % P == l % P)).astype(np.float32)
    e = np.arange(E)[:, None]
    rr = np.arange(R)[None, :]
    poolL = ((rr % (E * P)) // P == e).astype(np.float32)
    ll = np.arange(HW)[:, None]
    jj = np.arange(NPATCH)[None, :]
    poolR = ((ll // (P * W) == jj // WP)
             & ((ll % W) // P == jj % WP)).astype(np.float32)
    return mask, poolL, poolR


def kernel(x, spatial_shape, gate_w):
    del spatial_shape
    xr = x.reshape(B, C, HW)
    Wm = jnp.transpose(gate_w, (2, 0, 3, 1)).reshape(R, C)
    mask, poolL, poolR = _constants()

    out = pl.pallas_call(
        _router_kernel,
        grid=(B // G,),
        in_specs=[
            pl.BlockSpec((G, C, HW), lambda b: (b, 0, 0)),
            pl.BlockSpec((R, C), lambda b: (0, 0)),
            pl.BlockSpec((R, HW), lambda b: (0, 0)),
            pl.BlockSpec((E, R), lambda b: (0, 0)),
            pl.BlockSpec((HW, NPATCH), lambda b: (0, 0)),
        ],
        out_specs=[
            pl.BlockSpec((G, NPATCH, K), lambda b: (b, 0, 0)),
            pl.BlockSpec((G, NPATCH, K), lambda b: (b, 0, 0)),
            pl.BlockSpec((G, NPATCH, E), lambda b: (b, 0, 0)),
        ],
        out_shape=[
            jax.ShapeDtypeStruct((B, NPATCH, K), jnp.float32),
            jax.ShapeDtypeStruct((B, NPATCH, K), jnp.int32),
            jax.ShapeDtypeStruct((B, NPATCH, E), jnp.float32),
        ],
    )(xr, Wm, jnp.asarray(mask), jnp.asarray(poolL), jnp.asarray(poolR))

    expert_weights, expert_indices, router_logits = out
    return expert_weights, expert_indices, router_logits
